# baseline (speedup 1.0000x reference)
import functools

import jax
import jax.numpy as jnp
from jax import lax
from jax.experimental import pallas as pl
from jax.experimental.pallas import tpu as pltpu
from jax.experimental.pallas import tpu_sc as plsc

NC = 2
NS = 16
NW = NC * NS
L = 16
CH = 128
BLKC = 16


def _agg_call(N, D, NCH, split=None):
    if split is None:
        split = (NCH, NCH)
    NCH0, NCH1 = split
    assert NCH0 % BLKC == 0 and NCH1 % BLKC == 0 and NCH0 + NCH1 == 2 * NCH
    n_acc = -(-(N + 1) // 128) * 128
    rows_per = n_acc // NS
    mesh = plsc.VectorSubcoreMesh(core_axis_name="c", subcore_axis_name="s")

    CE = CH // 2
    B2 = 2 * BLKC
    NSLOT = 4

    @functools.partial(
        pl.kernel,
        out_type=jax.ShapeDtypeStruct((NC, n_acc, D), jnp.float32),
        mesh=mesh,
        scratch_types=[
            pltpu.VMEM((B2, CE), jnp.int32),
            pltpu.VMEM((B2, CE), jnp.int32),
            pltpu.VMEM((CE, D), jnp.float32),
            pltpu.VMEM((CE, D), jnp.float32),
            pltpu.VMEM((CE, D), jnp.float32),
            pltpu.VMEM((CE, D), jnp.float32),
            pltpu.VMEM_SHARED((n_acc, D), jnp.float32),
            pltpu.SemaphoreType.DMA,
            pltpu.SemaphoreType.DMA,
            pltpu.SemaphoreType.DMA,
            pltpu.SemaphoreType.DMA,
        ],
    )
    def agg(x_hbm, src_hbm, tgt_hbm, zero_hbm, part_hbm,
            src_v, tgt_v, buf0, buf1, buf2, buf3, acc,
            sem0, sem1, sem2, sem3):
        c = lax.axis_index("c")
        s = lax.axis_index("s")
        bufs = (buf0, buf1, buf2, buf3)
        sems = (sem0, sem1, sem2, sem3)
        r0 = s * rows_per
        pltpu.sync_copy(zero_hbm.at[pl.ds(r0, rows_per)],
                        acc.at[pl.ds(r0, rows_per)])
        plsc.subcore_barrier()

        chunk0 = jnp.where(c == 0, s * NCH0, NS * NCH0 + s * NCH1)
        nblk = jnp.where(c == 0, NCH0 // BLKC, NCH1 // BLKC)

        @pl.loop(0, nblk)
        def _(ib):
            hb = (chunk0 + ib * BLKC) * 2
            pltpu.sync_copy(src_hbm.at[pl.ds(hb, B2)], src_v)
            pltpu.sync_copy(tgt_hbm.at[pl.ds(hb, B2)], tgt_v)
            for bi in range(NSLOT):
                pltpu.async_copy(x_hbm.at[src_v.at[bi]], bufs[bi], sems[bi])

            @pl.loop(0, B2 // NSLOT - 1)
            def _(i):
                for bi in range(NSLOT):
                    k = i * NSLOT + bi
                    pltpu.make_async_copy(x_hbm.at[src_v.at[k]], bufs[bi],
                                          sems[bi]).wait()
                    pltpu.sync_copy(bufs[bi], acc.at[tgt_v.at[k]], add=True)
                    pltpu.async_copy(x_hbm.at[src_v.at[k + NSLOT]],
                                     bufs[bi], sems[bi])

            for bi in range(NSLOT):
                k = B2 - NSLOT + bi
                pltpu.make_async_copy(x_hbm.at[src_v.at[k]], bufs[bi],
                                      sems[bi]).wait()
                pltpu.sync_copy(bufs[bi], acc.at[tgt_v.at[k]], add=True)

        plsc.subcore_barrier()
        pltpu.sync_copy(acc.at[pl.ds(r0, rows_per)],
                        part_hbm.at[c].at[pl.ds(r0, rows_per)])

    return agg


def _dot_call(N, D, NCH, NP):
    NSL = 4
    CE = CH // NSL
    B4 = NSL * BLKC
    NB = NCH // BLKC
    rows_stage = NP // NS
    mesh = plsc.VectorSubcoreMesh(core_axis_name="c", subcore_axis_name="s")

    @functools.partial(
        pl.kernel,
        out_type=jax.ShapeDtypeStruct((NW * NCH * CH * L,), jnp.float32),
        mesh=mesh,
        scratch_types=[
            pltpu.VMEM((BLKC, CH), jnp.int32),
            pltpu.VMEM((BLKC, CH), jnp.int32),
            pltpu.VMEM_SHARED((NP, D), jnp.float32),
        ] + [pltpu.VMEM((CE, D), jnp.float32)] * (2 * NSL)
          + [pltpu.VMEM((CE * L,), jnp.float32)] * NSL
          + [pltpu.SemaphoreType.DMA] * (2 * NSL),
    )
    def dot(h_hbm, src_hbm, tgt_hbm, out_hbm, src_v, tgt_v, hsp, *rest):
        bs = rest[0:NSL]
        bt = rest[NSL:2 * NSL]
        pa = rest[2 * NSL:3 * NSL]
        sems = rest[3 * NSL:4 * NSL]
        semo = rest[4 * NSL:5 * NSL]
        c = lax.axis_index("c")
        s = lax.axis_index("s")
        w = c * NS + s

        r0 = s * rows_stage
        pltpu.sync_copy(h_hbm.at[pl.ds(r0, rows_stage)],
                        hsp.at[pl.ds(r0, rows_stage)])
        plsc.subcore_barrier()

        def idx_ref(v, hh):
            return v.at[hh // NSL, pl.ds((hh % NSL) * CE, CE)]

        def out_ref(ib, hh):
            off = (w * NCH * CH + ib * BLKC * CH + hh * CE) * L
            return out_hbm.at[pl.ds(off, CE * L)]

        def fire(hh, bi):
            pltpu.async_copy(hsp.at[idx_ref(src_v, hh)], bs[bi], sems[bi])
            pltpu.async_copy(hsp.at[idx_ref(tgt_v, hh)], bt[bi], sems[bi])

        def compute(hh, bi):
            pltpu.make_async_copy(hsp.at[idx_ref(src_v, hh)], bs[bi],
                                  sems[bi]).wait()
            pltpu.make_async_copy(hsp.at[idx_ref(tgt_v, hh)], bt[bi],
                                  sems[bi]).wait()

            @pl.loop(0, CE, unroll=4)
            def _(e):
                acc = bs[bi][e, pl.ds(0, L)] * bt[bi][e, pl.ds(0, L)]
                for k in range(1, D // L):
                    acc = acc + (bs[bi][e, pl.ds(k * L, L)] *
                                 bt[bi][e, pl.ds(k * L, L)])
                pa[bi][pl.ds(e * L, L)] = acc

        @pl.loop(0, NB)
        def _(ib):
            b0 = (w * NCH) + ib * BLKC
            pltpu.sync_copy(src_hbm.at[pl.ds(b0, BLKC)], src_v)
            pltpu.sync_copy(tgt_hbm.at[pl.ds(b0, BLKC)], tgt_v)
            for bi in range(NSL):
                fire(bi, bi)
            for bi in range(NSL):
                compute(bi, bi)
                fire(bi + NSL, bi)
                pltpu.async_copy(pa[bi], out_ref(ib, bi), semo[bi])

            @pl.loop(1, B4 // NSL - 1)
            def _(i):
                for bi in range(NSL):
                    hh = i * NSL + bi
                    pltpu.make_async_copy(pa[bi], out_ref(ib, hh),
                                          semo[bi]).wait()
                    compute(hh, bi)
                    fire(hh + NSL, bi)
                    pltpu.async_copy(pa[bi], out_ref(ib, hh), semo[bi])

            for bi in range(NSL):
                hh = B4 - NSL + bi
                pltpu.make_async_copy(pa[bi], out_ref(ib, hh),
                                      semo[bi]).wait()
                compute(hh, bi)
                pltpu.async_copy(pa[bi], out_ref(ib, hh), semo[bi])
            for bi in range(NSL):
                pltpu.make_async_copy(pa[bi], out_ref(ib, 0),
                                      semo[bi]).wait()

    return dot


def _reduce16(p):
    M = p.shape[0] // 128
    BLK = 4096
    EPR = 128 // L

    def red(pr, outr):
        x = pr[...]
        cols = [jnp.sum(x[:, e * L:(e + 1) * L], axis=1, keepdims=True)
                for e in range(EPR)]
        outr[...] = jnp.concatenate(cols, axis=1)

    return pl.pallas_call(
        red,
        grid=(M // BLK,),
        in_specs=[pl.BlockSpec((BLK, 128), lambda i: (i, 0))],
        out_specs=pl.BlockSpec((BLK, EPR), lambda i: (i, 0)),
        out_shape=jax.ShapeDtypeStruct((M, EPR), jnp.float32),
    )(p.reshape(M, 128))


def _dense(p0, p1, x, W_neigh, W_self, b2):
    N, D = x.shape
    BLK = 2000

    def mm(p0r, p1r, xr, wn, ws, br, hr):
        agg = p0r[...] + p1r[...]
        acc = jnp.dot(agg, wn[...], preferred_element_type=jnp.float32)
        acc = acc + jnp.dot(xr[...], ws[...], preferred_element_type=jnp.float32)
        hr[...] = jnp.maximum(acc + br[...], 0.0)

    row_spec = pl.BlockSpec((BLK, D), lambda i: (i, 0))
    w_spec = pl.BlockSpec((D, D), lambda i: (0, 0))
    return pl.pallas_call(
        mm,
        grid=(N // BLK,),
        in_specs=[row_spec, row_spec, row_spec, w_spec, w_spec,
                  pl.BlockSpec((1, D), lambda i: (0, 0))],
        out_specs=row_spec,
        out_shape=jax.ShapeDtypeStruct((N, D), jnp.float32),
    )(p0, p1, x, W_neigh, W_self, b2)


def kernel(x, edge_index, W_neigh, W_self, b):
    N, D = x.shape
    E = edge_index.shape[1]
    n_acc = -(-(N + 1) // 128) * 128
    NCH = -(-E // (CH * NW))
    NCH = -(-NCH // BLKC) * BLKC
    e_pad = NW * NCH * CH
    pad = e_pad - E

    src = edge_index[0]
    tgt = edge_index[1]
    src_p = jnp.concatenate(
        [src, jnp.zeros((pad,), jnp.int32)]).reshape(NW * NCH, CH)
    pad_tgt = N + jnp.arange(pad, dtype=jnp.int32) % (n_acc - N)
    tgt_a = jnp.concatenate([tgt, pad_tgt]).reshape(NW * NCH, CH)
    tgt_c = jnp.concatenate(
        [tgt, jnp.zeros((pad,), jnp.int32)]).reshape(NW * NCH, CH)
    zeros = jnp.zeros((n_acc, D), jnp.float32)

    split = (48 * (NCH // 80), 112 * (NCH // 80)) if NCH % 80 == 0 else None
    parts = _agg_call(N, D, NCH, split)(x, src_p.reshape(-1, CH // 2),
                                        tgt_a.reshape(-1, CH // 2), zeros)
    h = _dense(parts[0, :N], parts[1, :N], x, W_neigh, W_self,
               b.reshape(1, D))
    h_pad = jnp.concatenate([h, jnp.zeros((n_acc - N, D), jnp.float32)])
    partial16 = _dot_call(N, D, NCH, n_acc)(h_pad, src_p, tgt_c)
    scores = _reduce16(partial16)
    return scores.reshape(-1)[:E]


# speedup vs baseline: 1.0265x; 1.0265x over previous
import functools

import jax
import jax.numpy as jnp
from jax import lax
from jax.experimental import pallas as pl
from jax.experimental.pallas import tpu as pltpu
from jax.experimental.pallas import tpu_sc as plsc

NC = 2
NS = 16
NW = NC * NS
L = 16
CH = 128
BLKC = 16


def _agg_call(N, D, NCH, split=None):
    if split is None:
        split = (NCH, NCH)
    NCH0, NCH1 = split
    assert NCH0 % BLKC == 0 and NCH1 % BLKC == 0 and NCH0 + NCH1 == 2 * NCH
    n_acc = -(-(N + 1) // 128) * 128
    rows_per = n_acc // NS
    mesh = plsc.VectorSubcoreMesh(core_axis_name="c", subcore_axis_name="s")

    CE = CH // 2
    B2 = 2 * BLKC
    NSLOT = 4

    @functools.partial(
        pl.kernel,
        out_type=jax.ShapeDtypeStruct((NC, n_acc, D), jnp.float32),
        mesh=mesh,
        scratch_types=[
            pltpu.VMEM((B2, CE), jnp.int32),
            pltpu.VMEM((B2, CE), jnp.int32),
            pltpu.VMEM((CE, D), jnp.float32),
            pltpu.VMEM((CE, D), jnp.float32),
            pltpu.VMEM((CE, D), jnp.float32),
            pltpu.VMEM((CE, D), jnp.float32),
            pltpu.VMEM_SHARED((n_acc, D), jnp.float32),
            pltpu.SemaphoreType.DMA,
            pltpu.SemaphoreType.DMA,
            pltpu.SemaphoreType.DMA,
            pltpu.SemaphoreType.DMA,
        ],
    )
    def agg(x_hbm, src_hbm, tgt_hbm, zero_hbm, part_hbm,
            src_v, tgt_v, buf0, buf1, buf2, buf3, acc,
            sem0, sem1, sem2, sem3):
        c = lax.axis_index("c")
        s = lax.axis_index("s")
        bufs = (buf0, buf1, buf2, buf3)
        sems = (sem0, sem1, sem2, sem3)
        r0 = s * rows_per
        pltpu.sync_copy(zero_hbm.at[pl.ds(r0, rows_per)],
                        acc.at[pl.ds(r0, rows_per)])
        plsc.subcore_barrier()

        chunk0 = jnp.where(c == 0, s * NCH0, NS * NCH0 + s * NCH1)
        nblk = jnp.where(c == 0, NCH0 // BLKC, NCH1 // BLKC)

        @pl.loop(0, nblk)
        def _(ib):
            hb = (chunk0 + ib * BLKC) * 2
            pltpu.sync_copy(src_hbm.at[pl.ds(hb, B2)], src_v)
            pltpu.sync_copy(tgt_hbm.at[pl.ds(hb, B2)], tgt_v)
            for bi in range(NSLOT):
                pltpu.async_copy(x_hbm.at[src_v.at[bi]], bufs[bi], sems[bi])

            @pl.loop(0, B2 // NSLOT - 1)
            def _(i):
                for bi in range(NSLOT):
                    k = i * NSLOT + bi
                    pltpu.make_async_copy(x_hbm.at[src_v.at[k]], bufs[bi],
                                          sems[bi]).wait()
                    pltpu.sync_copy(bufs[bi], acc.at[tgt_v.at[k]], add=True)
                    pltpu.async_copy(x_hbm.at[src_v.at[k + NSLOT]],
                                     bufs[bi], sems[bi])

            for bi in range(NSLOT):
                k = B2 - NSLOT + bi
                pltpu.make_async_copy(x_hbm.at[src_v.at[k]], bufs[bi],
                                      sems[bi]).wait()
                pltpu.sync_copy(bufs[bi], acc.at[tgt_v.at[k]], add=True)

        plsc.subcore_barrier()
        pltpu.sync_copy(acc.at[pl.ds(r0, rows_per)],
                        part_hbm.at[c].at[pl.ds(r0, rows_per)])

    return agg


def _dot_call(N, D, NCH, NP):
    NSL = 4
    CE = CH // NSL
    B4 = NSL * BLKC
    NB = NCH // BLKC
    rows_stage = NP // NS
    mesh = plsc.VectorSubcoreMesh(core_axis_name="c", subcore_axis_name="s")

    @functools.partial(
        pl.kernel,
        out_type=jax.ShapeDtypeStruct((NW * NCH * CH * L,), jnp.float32),
        mesh=mesh,
        scratch_types=[
            pltpu.VMEM((BLKC, CH), jnp.int32),
            pltpu.VMEM((BLKC, CH), jnp.int32),
            pltpu.VMEM_SHARED((NP, D), jnp.float32),
        ] + [pltpu.VMEM((CE, D), jnp.float32)] * (2 * NSL)
          + [pltpu.VMEM((CE * L,), jnp.float32)] * NSL
          + [pltpu.SemaphoreType.DMA] * (2 * NSL),
    )
    def dot(h_hbm, src_hbm, tgt_hbm, out_hbm, src_v, tgt_v, hsp, *rest):
        bs = rest[0:NSL]
        bt = rest[NSL:2 * NSL]
        pa = rest[2 * NSL:3 * NSL]
        sems = rest[3 * NSL:4 * NSL]
        semo = rest[4 * NSL:5 * NSL]
        c = lax.axis_index("c")
        s = lax.axis_index("s")
        w = c * NS + s

        r0 = s * rows_stage
        pltpu.sync_copy(h_hbm.at[pl.ds(r0, rows_stage)],
                        hsp.at[pl.ds(r0, rows_stage)])
        plsc.subcore_barrier()

        def idx_ref(v, hh):
            return v.at[hh // NSL, pl.ds((hh % NSL) * CE, CE)]

        def out_ref(ib, hh):
            off = (w * NCH * CH + ib * BLKC * CH + hh * CE) * L
            return out_hbm.at[pl.ds(off, CE * L)]

        def fire(hh, bi):
            pltpu.async_copy(hsp.at[idx_ref(src_v, hh)], bs[bi], sems[bi])
            pltpu.async_copy(hsp.at[idx_ref(tgt_v, hh)], bt[bi], sems[bi])

        def compute(hh, bi):
            pltpu.make_async_copy(hsp.at[idx_ref(src_v, hh)], bs[bi],
                                  sems[bi]).wait()
            pltpu.make_async_copy(hsp.at[idx_ref(tgt_v, hh)], bt[bi],
                                  sems[bi]).wait()

            @pl.loop(0, CE, unroll=4)
            def _(e):
                acc = bs[bi][e, pl.ds(0, L)] * bt[bi][e, pl.ds(0, L)]
                for k in range(1, D // L):
                    acc = acc + (bs[bi][e, pl.ds(k * L, L)] *
                                 bt[bi][e, pl.ds(k * L, L)])
                pa[bi][pl.ds(e * L, L)] = acc

        @pl.loop(0, NB)
        def _(ib):
            b0 = (w * NCH) + ib * BLKC
            pltpu.sync_copy(src_hbm.at[pl.ds(b0, BLKC)], src_v)
            pltpu.sync_copy(tgt_hbm.at[pl.ds(b0, BLKC)], tgt_v)
            for bi in range(NSL):
                fire(bi, bi)
            for bi in range(NSL):
                compute(bi, bi)
                fire(bi + NSL, bi)
                pltpu.async_copy(pa[bi], out_ref(ib, bi), semo[bi])

            @pl.loop(1, B4 // NSL - 1)
            def _(i):
                for bi in range(NSL):
                    hh = i * NSL + bi
                    pltpu.make_async_copy(pa[bi], out_ref(ib, hh),
                                          semo[bi]).wait()
                    compute(hh, bi)
                    fire(hh + NSL, bi)
                    pltpu.async_copy(pa[bi], out_ref(ib, hh), semo[bi])

            for bi in range(NSL):
                hh = B4 - NSL + bi
                pltpu.make_async_copy(pa[bi], out_ref(ib, hh),
                                      semo[bi]).wait()
                compute(hh, bi)
                pltpu.async_copy(pa[bi], out_ref(ib, hh), semo[bi])
            for bi in range(NSL):
                pltpu.make_async_copy(pa[bi], out_ref(ib, 0),
                                      semo[bi]).wait()

    return dot


def _reduce16(p):
    M = p.shape[0] // 128
    BLK = 4096
    EPR = 128 // L

    def red(pr, outr):
        x = pr[...]
        cols = [jnp.sum(x[:, e * L:(e + 1) * L], axis=1, keepdims=True)
                for e in range(EPR)]
        outr[...] = jnp.concatenate(cols, axis=1)

    return pl.pallas_call(
        red,
        grid=(M // BLK,),
        in_specs=[pl.BlockSpec((BLK, 128), lambda i: (i, 0))],
        out_specs=pl.BlockSpec((BLK, EPR), lambda i: (i, 0)),
        out_shape=jax.ShapeDtypeStruct((M, EPR), jnp.float32),
    )(p.reshape(M, 128))


def _dense(p0, p1, x, W_neigh, W_self, b2):
    N, D = x.shape
    BLK = 2000

    def mm(p0r, p1r, xr, wn, ws, br, hr):
        agg = p0r[...] + p1r[...]
        acc = jnp.dot(agg, wn[...], preferred_element_type=jnp.float32)
        acc = acc + jnp.dot(xr[...], ws[...], preferred_element_type=jnp.float32)
        hr[...] = jnp.maximum(acc + br[...], 0.0)

    row_spec = pl.BlockSpec((BLK, D), lambda i: (i, 0))
    w_spec = pl.BlockSpec((D, D), lambda i: (0, 0))
    return pl.pallas_call(
        mm,
        grid=(N // BLK,),
        in_specs=[row_spec, row_spec, row_spec, w_spec, w_spec,
                  pl.BlockSpec((1, D), lambda i: (0, 0))],
        out_specs=row_spec,
        out_shape=jax.ShapeDtypeStruct((N, D), jnp.float32),
    )(p0, p1, x, W_neigh, W_self, b2)


def kernel(x, edge_index, W_neigh, W_self, b):
    N, D = x.shape
    E = edge_index.shape[1]
    n_acc = -(-(N + 1) // 128) * 128
    NCH = -(-E // (CH * NW))
    NCH = -(-NCH // BLKC) * BLKC
    e_pad = NW * NCH * CH
    pad = e_pad - E

    src = edge_index[0]
    tgt = edge_index[1]
    src_p = jnp.concatenate(
        [src, jnp.zeros((pad,), jnp.int32)]).reshape(NW * NCH, CH)
    pad_tgt = N + jnp.arange(pad, dtype=jnp.int32) % (n_acc - N)
    tgt_a = jnp.concatenate([tgt, pad_tgt]).reshape(NW * NCH, CH)
    tgt_c = jnp.concatenate(
        [tgt, jnp.zeros((pad,), jnp.int32)]).reshape(NW * NCH, CH)
    zeros = jnp.zeros((n_acc, D), jnp.float32)

    split = (112 * (NCH // 80), 48 * (NCH // 80)) if NCH % 80 == 0 else None
    parts = _agg_call(N, D, NCH, split)(x, src_p.reshape(-1, CH // 2),
                                        tgt_a.reshape(-1, CH // 2), zeros)
    h = _dense(parts[0, :N], parts[1, :N], x, W_neigh, W_self,
               b.reshape(1, D))
    h_pad = jnp.concatenate([h, jnp.zeros((n_acc - N, D), jnp.float32)])
    partial16 = _dot_call(N, D, NCH, n_acc)(h_pad, src_p, tgt_c)
    scores = _reduce16(partial16)
    return scores.reshape(-1)[:E]


# speedup vs baseline: 1.0439x; 1.0169x over previous
import functools

import jax
import jax.numpy as jnp
from jax import lax
from jax.experimental import pallas as pl
from jax.experimental.pallas import tpu as pltpu
from jax.experimental.pallas import tpu_sc as plsc

NC = 2
NS = 16
NW = NC * NS
L = 16
CH = 128
BLKC = 16


def _agg_call(N, D, NCH, split=None):
    if split is None:
        split = (NCH, NCH)
    NCH0, NCH1 = split
    assert NCH0 % BLKC == 0 and NCH1 % BLKC == 0 and NCH0 + NCH1 == 2 * NCH
    n_acc = -(-(N + 1) // 128) * 128
    rows_per = n_acc // NS
    mesh = plsc.VectorSubcoreMesh(core_axis_name="c", subcore_axis_name="s")

    CE = CH // 2
    B2 = 2 * BLKC
    NSLOT = 4

    @functools.partial(
        pl.kernel,
        out_type=jax.ShapeDtypeStruct((NC, n_acc, D), jnp.float32),
        mesh=mesh,
        scratch_types=[
            pltpu.VMEM((B2, CE), jnp.int32),
            pltpu.VMEM((B2, CE), jnp.int32),
            pltpu.VMEM((CE, D), jnp.float32),
            pltpu.VMEM((CE, D), jnp.float32),
            pltpu.VMEM((CE, D), jnp.float32),
            pltpu.VMEM((CE, D), jnp.float32),
            pltpu.VMEM_SHARED((n_acc, D), jnp.float32),
            pltpu.SemaphoreType.DMA,
            pltpu.SemaphoreType.DMA,
            pltpu.SemaphoreType.DMA,
            pltpu.SemaphoreType.DMA,
        ],
    )
    def agg(x_hbm, src_hbm, tgt_hbm, zero_hbm, part_hbm,
            src_v, tgt_v, buf0, buf1, buf2, buf3, acc,
            sem0, sem1, sem2, sem3):
        c = lax.axis_index("c")
        s = lax.axis_index("s")
        bufs = (buf0, buf1, buf2, buf3)
        sems = (sem0, sem1, sem2, sem3)
        r0 = s * rows_per
        pltpu.sync_copy(zero_hbm.at[pl.ds(r0, rows_per)],
                        acc.at[pl.ds(r0, rows_per)])
        plsc.subcore_barrier()

        chunk0 = jnp.where(c == 0, s * NCH0, NS * NCH0 + s * NCH1)
        nblk = jnp.where(c == 0, NCH0 // BLKC, NCH1 // BLKC)

        @pl.loop(0, nblk)
        def _(ib):
            hb = (chunk0 + ib * BLKC) * 2
            pltpu.sync_copy(src_hbm.at[pl.ds(hb, B2)], src_v)
            pltpu.sync_copy(tgt_hbm.at[pl.ds(hb, B2)], tgt_v)
            for bi in range(NSLOT):
                pltpu.async_copy(x_hbm.at[src_v.at[bi]], bufs[bi], sems[bi])

            @pl.loop(0, B2 // NSLOT - 1)
            def _(i):
                for bi in range(NSLOT):
                    k = i * NSLOT + bi
                    pltpu.make_async_copy(x_hbm.at[src_v.at[k]], bufs[bi],
                                          sems[bi]).wait()
                    pltpu.sync_copy(bufs[bi], acc.at[tgt_v.at[k]], add=True)
                    pltpu.async_copy(x_hbm.at[src_v.at[k + NSLOT]],
                                     bufs[bi], sems[bi])

            for bi in range(NSLOT):
                k = B2 - NSLOT + bi
                pltpu.make_async_copy(x_hbm.at[src_v.at[k]], bufs[bi],
                                      sems[bi]).wait()
                pltpu.sync_copy(bufs[bi], acc.at[tgt_v.at[k]], add=True)

        plsc.subcore_barrier()
        pltpu.sync_copy(acc.at[pl.ds(r0, rows_per)],
                        part_hbm.at[c].at[pl.ds(r0, rows_per)])

    return agg


def _dot_call(N, D, NCH, NP):
    NSL = 4
    CE = CH // NSL
    B4 = NSL * BLKC
    NB = NCH // BLKC
    rows_stage = NP // NS
    mesh = plsc.VectorSubcoreMesh(core_axis_name="c", subcore_axis_name="s")

    @functools.partial(
        pl.kernel,
        out_type=jax.ShapeDtypeStruct((NW * NCH * CH * L,), jnp.float32),
        mesh=mesh,
        scratch_types=[
            pltpu.VMEM((BLKC, CH), jnp.int32),
            pltpu.VMEM((BLKC, CH), jnp.int32),
            pltpu.VMEM_SHARED((NP, D), jnp.float32),
        ] + [pltpu.VMEM((CE, D), jnp.float32)] * (2 * NSL)
          + [pltpu.VMEM((CE * L,), jnp.float32)] * NSL
          + [pltpu.SemaphoreType.DMA] * (2 * NSL),
    )
    def dot(h_hbm, src_hbm, tgt_hbm, out_hbm, src_v, tgt_v, hsp, *rest):
        bs = rest[0:NSL]
        bt = rest[NSL:2 * NSL]
        pa = rest[2 * NSL:3 * NSL]
        sems = rest[3 * NSL:4 * NSL]
        semo = rest[4 * NSL:5 * NSL]
        c = lax.axis_index("c")
        s = lax.axis_index("s")
        w = c * NS + s

        r0 = s * rows_stage
        pltpu.sync_copy(h_hbm.at[pl.ds(r0, rows_stage)],
                        hsp.at[pl.ds(r0, rows_stage)])
        plsc.subcore_barrier()

        def idx_ref(v, hh):
            return v.at[hh // NSL, pl.ds((hh % NSL) * CE, CE)]

        def out_ref(ib, hh):
            off = (w * NCH * CH + ib * BLKC * CH + hh * CE) * L
            return out_hbm.at[pl.ds(off, CE * L)]

        def fire(hh, bi):
            pltpu.async_copy(hsp.at[idx_ref(src_v, hh)], bs[bi], sems[bi])
            pltpu.async_copy(hsp.at[idx_ref(tgt_v, hh)], bt[bi], sems[bi])

        def compute(hh, bi):
            pltpu.make_async_copy(hsp.at[idx_ref(src_v, hh)], bs[bi],
                                  sems[bi]).wait()
            pltpu.make_async_copy(hsp.at[idx_ref(tgt_v, hh)], bt[bi],
                                  sems[bi]).wait()

            @pl.loop(0, CE, unroll=4)
            def _(e):
                acc = bs[bi][e, pl.ds(0, L)] * bt[bi][e, pl.ds(0, L)]
                for k in range(1, D // L):
                    acc = acc + (bs[bi][e, pl.ds(k * L, L)] *
                                 bt[bi][e, pl.ds(k * L, L)])
                pa[bi][pl.ds(e * L, L)] = acc

        @pl.loop(0, NB)
        def _(ib):
            b0 = (w * NCH) + ib * BLKC
            pltpu.sync_copy(src_hbm.at[pl.ds(b0, BLKC)], src_v)
            pltpu.sync_copy(tgt_hbm.at[pl.ds(b0, BLKC)], tgt_v)
            for bi in range(NSL):
                fire(bi, bi)
            for bi in range(NSL):
                compute(bi, bi)
                fire(bi + NSL, bi)
                pltpu.async_copy(pa[bi], out_ref(ib, bi), semo[bi])

            @pl.loop(1, B4 // NSL - 1)
            def _(i):
                for bi in range(NSL):
                    hh = i * NSL + bi
                    pltpu.make_async_copy(pa[bi], out_ref(ib, hh),
                                          semo[bi]).wait()
                    compute(hh, bi)
                    fire(hh + NSL, bi)
                    pltpu.async_copy(pa[bi], out_ref(ib, hh), semo[bi])

            for bi in range(NSL):
                hh = B4 - NSL + bi
                pltpu.make_async_copy(pa[bi], out_ref(ib, hh),
                                      semo[bi]).wait()
                compute(hh, bi)
                pltpu.async_copy(pa[bi], out_ref(ib, hh), semo[bi])
            for bi in range(NSL):
                pltpu.make_async_copy(pa[bi], out_ref(ib, 0),
                                      semo[bi]).wait()

    return dot


def _reduce16(p):
    M = p.shape[0] // 128
    BLK = 4096
    EPR = 128 // L

    def red(pr, outr):
        x = pr[...]
        cols = [jnp.sum(x[:, e * L:(e + 1) * L], axis=1, keepdims=True)
                for e in range(EPR)]
        outr[...] = jnp.concatenate(cols, axis=1)

    return pl.pallas_call(
        red,
        grid=(M // BLK,),
        in_specs=[pl.BlockSpec((BLK, 128), lambda i: (i, 0))],
        out_specs=pl.BlockSpec((BLK, EPR), lambda i: (i, 0)),
        out_shape=jax.ShapeDtypeStruct((M, EPR), jnp.float32),
    )(p.reshape(M, 128))


def _dense(parts, x, W_neigh, W_self, b2, NP):
    N, D = x.shape
    BLK = 2000

    def mm(pr, xr, wn, ws, br, hr):
        agg = pr[0] + pr[1]
        acc = jnp.dot(agg, wn[...], preferred_element_type=jnp.float32)
        acc = acc + jnp.dot(xr[...], ws[...], preferred_element_type=jnp.float32)
        hr[...] = jnp.maximum(acc + br[...], 0.0)

    row_spec = pl.BlockSpec((BLK, D), lambda i: (i, 0))
    w_spec = pl.BlockSpec((D, D), lambda i: (0, 0))
    return pl.pallas_call(
        mm,
        grid=(N // BLK,),
        in_specs=[pl.BlockSpec((2, BLK, D), lambda i: (0, i, 0)), row_spec,
                  w_spec, w_spec, pl.BlockSpec((1, D), lambda i: (0, 0))],
        out_specs=row_spec,
        out_shape=jax.ShapeDtypeStruct((NP, D), jnp.float32),
    )(parts, x, W_neigh, W_self, b2)


def kernel(x, edge_index, W_neigh, W_self, b):
    N, D = x.shape
    E = edge_index.shape[1]
    n_acc = -(-(N + 1) // 128) * 128
    NCH = -(-E // (CH * NW))
    NCH = -(-NCH // BLKC) * BLKC
    e_pad = NW * NCH * CH
    pad = e_pad - E

    src = edge_index[0]
    tgt = edge_index[1]
    src_p = jnp.concatenate(
        [src, jnp.zeros((pad,), jnp.int32)]).reshape(NW * NCH, CH)
    pad_tgt = N + jnp.arange(pad, dtype=jnp.int32) % (n_acc - N)
    tgt_a = jnp.concatenate([tgt, pad_tgt]).reshape(NW * NCH, CH)
    tgt_c = jnp.concatenate(
        [tgt, jnp.zeros((pad,), jnp.int32)]).reshape(NW * NCH, CH)
    zeros = jnp.zeros((n_acc, D), jnp.float32)

    split = (112 * (NCH // 80), 48 * (NCH // 80)) if NCH % 80 == 0 else None
    parts = _agg_call(N, D, NCH, split)(x, src_p.reshape(-1, CH // 2),
                                        tgt_a.reshape(-1, CH // 2), zeros)
    h_pad = _dense(parts, x, W_neigh, W_self, b.reshape(1, D), n_acc)
    partial16 = _dot_call(N, D, NCH, n_acc)(h_pad, src_p, tgt_c)
    scores = _reduce16(partial16)
    return scores.reshape(-1)[:E]


# speedup vs baseline: 1.1547x; 1.1062x over previous
import functools

import jax
import jax.numpy as jnp
from jax import lax
from jax.experimental import pallas as pl
from jax.experimental.pallas import tpu as pltpu
from jax.experimental.pallas import tpu_sc as plsc

NC = 2
NS = 16
NW = NC * NS
L = 16
CH = 128
BLKC = 16


def _agg_call(N, D, NCH, split=None):
    if split is None:
        split = (NCH, NCH)
    NCH0, NCH1 = split
    assert NCH0 % BLKC == 0 and NCH1 % BLKC == 0 and NCH0 + NCH1 == 2 * NCH
    n_acc = -(-(N + 1) // 128) * 128
    rows_per = n_acc // NS
    mesh = plsc.VectorSubcoreMesh(core_axis_name="c", subcore_axis_name="s")

    CE = CH // 2
    B2 = 2 * BLKC
    NSLOT = 4

    @functools.partial(
        pl.kernel,
        out_type=jax.ShapeDtypeStruct((NC, n_acc, D), jnp.float32),
        mesh=mesh,
        scratch_types=[
            pltpu.VMEM((B2, CE), jnp.int32),
            pltpu.VMEM((B2, CE), jnp.int32),
            pltpu.VMEM((CE, D), jnp.float32),
            pltpu.VMEM((CE, D), jnp.float32),
            pltpu.VMEM((CE, D), jnp.float32),
            pltpu.VMEM((CE, D), jnp.float32),
            pltpu.VMEM_SHARED((n_acc, D), jnp.float32),
            pltpu.SemaphoreType.DMA,
            pltpu.SemaphoreType.DMA,
            pltpu.SemaphoreType.DMA,
            pltpu.SemaphoreType.DMA,
        ],
    )
    def agg(x_hbm, src_hbm, tgt_hbm, zero_hbm, part_hbm,
            src_v, tgt_v, buf0, buf1, buf2, buf3, acc,
            sem0, sem1, sem2, sem3):
        c = lax.axis_index("c")
        s = lax.axis_index("s")
        bufs = (buf0, buf1, buf2, buf3)
        sems = (sem0, sem1, sem2, sem3)
        r0 = s * rows_per
        pltpu.sync_copy(zero_hbm.at[pl.ds(r0, rows_per)],
                        acc.at[pl.ds(r0, rows_per)])
        plsc.subcore_barrier()

        chunk0 = jnp.where(c == 0, s * NCH0, NS * NCH0 + s * NCH1)
        nblk = jnp.where(c == 0, NCH0 // BLKC, NCH1 // BLKC)

        @pl.loop(0, nblk)
        def _(ib):
            hb = (chunk0 + ib * BLKC) * 2
            pltpu.sync_copy(src_hbm.at[pl.ds(hb, B2)], src_v)
            pltpu.sync_copy(tgt_hbm.at[pl.ds(hb, B2)], tgt_v)
            for bi in range(NSLOT):
                pltpu.async_copy(x_hbm.at[src_v.at[bi]], bufs[bi], sems[bi])

            @pl.loop(0, B2 // NSLOT - 1)
            def _(i):
                for bi in range(NSLOT):
                    k = i * NSLOT + bi
                    pltpu.make_async_copy(x_hbm.at[src_v.at[k]], bufs[bi],
                                          sems[bi]).wait()
                    pltpu.sync_copy(bufs[bi], acc.at[tgt_v.at[k]], add=True)
                    pltpu.async_copy(x_hbm.at[src_v.at[k + NSLOT]],
                                     bufs[bi], sems[bi])

            for bi in range(NSLOT):
                k = B2 - NSLOT + bi
                pltpu.make_async_copy(x_hbm.at[src_v.at[k]], bufs[bi],
                                      sems[bi]).wait()
                pltpu.sync_copy(bufs[bi], acc.at[tgt_v.at[k]], add=True)

        plsc.subcore_barrier()
        pltpu.sync_copy(acc.at[pl.ds(r0, rows_per)],
                        part_hbm.at[c].at[pl.ds(r0, rows_per)])

    return agg


def _dot_call(N, D, NCH, NP):
    NSL = 4
    CE = CH // NSL
    B4 = NSL * BLKC
    NB = NCH // BLKC
    rows_stage = NP // NS
    mesh = plsc.VectorSubcoreMesh(core_axis_name="c", subcore_axis_name="s")

    @functools.partial(
        pl.kernel,
        out_type=jax.ShapeDtypeStruct((NW * NCH * CH * L,), jnp.float32),
        mesh=mesh,
        scratch_types=[
            pltpu.VMEM((BLKC, CH), jnp.int32),
            pltpu.VMEM((BLKC, CH), jnp.int32),
            pltpu.VMEM_SHARED((NP, D), jnp.float32),
        ] + [pltpu.VMEM((CE, D), jnp.float32)] * (2 * NSL)
          + [pltpu.VMEM((CE * L,), jnp.float32)] * NSL
          + [pltpu.SemaphoreType.DMA] * (2 * NSL),
    )
    def dot(h_hbm, src_hbm, tgt_hbm, out_hbm, src_v, tgt_v, hsp, *rest):
        bs = rest[0:NSL]
        bt = rest[NSL:2 * NSL]
        pa = rest[2 * NSL:3 * NSL]
        sems = rest[3 * NSL:4 * NSL]
        semo = rest[4 * NSL:5 * NSL]
        c = lax.axis_index("c")
        s = lax.axis_index("s")
        w = c * NS + s

        r0 = s * rows_stage
        pltpu.sync_copy(h_hbm.at[pl.ds(r0, rows_stage)],
                        hsp.at[pl.ds(r0, rows_stage)])
        plsc.subcore_barrier()

        def idx_ref(v, hh):
            return v.at[hh // NSL, pl.ds((hh % NSL) * CE, CE)]

        def out_ref(ib, hh):
            off = (w * NCH * CH + ib * BLKC * CH + hh * CE) * L
            return out_hbm.at[pl.ds(off, CE * L)]

        def fire(hh, bi):
            pltpu.async_copy(hsp.at[idx_ref(src_v, hh)], bs[bi], sems[bi])
            pltpu.async_copy(hsp.at[idx_ref(tgt_v, hh)], bt[bi], sems[bi])

        def compute(hh, bi):
            pltpu.make_async_copy(hsp.at[idx_ref(src_v, hh)], bs[bi],
                                  sems[bi]).wait()
            pltpu.make_async_copy(hsp.at[idx_ref(tgt_v, hh)], bt[bi],
                                  sems[bi]).wait()

            @pl.loop(0, CE, unroll=4)
            def _(e):
                acc = bs[bi][e, pl.ds(0, L)] * bt[bi][e, pl.ds(0, L)]
                for k in range(1, D // L):
                    acc = acc + (bs[bi][e, pl.ds(k * L, L)] *
                                 bt[bi][e, pl.ds(k * L, L)])
                pa[bi][pl.ds(e * L, L)] = acc

        @pl.loop(0, NB)
        def _(ib):
            b0 = (w * NCH) + ib * BLKC
            pltpu.sync_copy(src_hbm.at[pl.ds(b0, BLKC)], src_v)
            pltpu.sync_copy(tgt_hbm.at[pl.ds(b0, BLKC)], tgt_v)
            for bi in range(NSL):
                fire(bi, bi)
            for bi in range(NSL):
                compute(bi, bi)
                fire(bi + NSL, bi)
                pltpu.async_copy(pa[bi], out_ref(ib, bi), semo[bi])

            @pl.loop(1, B4 // NSL - 1)
            def _(i):
                for bi in range(NSL):
                    hh = i * NSL + bi
                    pltpu.make_async_copy(pa[bi], out_ref(ib, hh),
                                          semo[bi]).wait()
                    compute(hh, bi)
                    fire(hh + NSL, bi)
                    pltpu.async_copy(pa[bi], out_ref(ib, hh), semo[bi])

            for bi in range(NSL):
                hh = B4 - NSL + bi
                pltpu.make_async_copy(pa[bi], out_ref(ib, hh),
                                      semo[bi]).wait()
                compute(hh, bi)
                pltpu.async_copy(pa[bi], out_ref(ib, hh), semo[bi])
            for bi in range(NSL):
                pltpu.make_async_copy(pa[bi], out_ref(ib, 0),
                                      semo[bi]).wait()

    return dot


def _reduce16(p):
    M = p.shape[0] // 128
    BLK = 4096
    EPR = 128 // L

    def red(pr, rr, outr):
        outr[...] = jnp.dot(pr[...], rr[...],
                            preferred_element_type=jnp.float32)

    R = (jnp.arange(128, dtype=jnp.int32)[:, None] // L
         == jnp.arange(EPR, dtype=jnp.int32)[None, :]).astype(jnp.float32)
    return pl.pallas_call(
        red,
        grid=(M // BLK,),
        in_specs=[pl.BlockSpec((BLK, 128), lambda i: (i, 0)),
                  pl.BlockSpec((128, EPR), lambda i: (0, 0))],
        out_specs=pl.BlockSpec((BLK, EPR), lambda i: (i, 0)),
        out_shape=jax.ShapeDtypeStruct((M, EPR), jnp.float32),
    )(p.reshape(M, 128), R)


def _dense(parts, x, W_neigh, W_self, b2, NP):
    N, D = x.shape
    BLK = 2000

    def mm(pr, xr, wn, ws, br, hr):
        agg = pr[0] + pr[1]
        acc = jnp.dot(agg, wn[...], preferred_element_type=jnp.float32)
        acc = acc + jnp.dot(xr[...], ws[...], preferred_element_type=jnp.float32)
        hr[...] = jnp.maximum(acc + br[...], 0.0)

    row_spec = pl.BlockSpec((BLK, D), lambda i: (i, 0))
    w_spec = pl.BlockSpec((D, D), lambda i: (0, 0))
    return pl.pallas_call(
        mm,
        grid=(N // BLK,),
        in_specs=[pl.BlockSpec((2, BLK, D), lambda i: (0, i, 0)), row_spec,
                  w_spec, w_spec, pl.BlockSpec((1, D), lambda i: (0, 0))],
        out_specs=row_spec,
        out_shape=jax.ShapeDtypeStruct((NP, D), jnp.float32),
    )(parts, x, W_neigh, W_self, b2)


def kernel(x, edge_index, W_neigh, W_self, b):
    N, D = x.shape
    E = edge_index.shape[1]
    n_acc = -(-(N + 1) // 128) * 128
    NCH = -(-E // (CH * NW))
    NCH = -(-NCH // BLKC) * BLKC
    e_pad = NW * NCH * CH
    pad = e_pad - E

    src = edge_index[0]
    tgt = edge_index[1]
    src_p = jnp.concatenate(
        [src, jnp.zeros((pad,), jnp.int32)]).reshape(NW * NCH, CH)
    pad_tgt = N + jnp.arange(pad, dtype=jnp.int32) % (n_acc - N)
    tgt_a = jnp.concatenate([tgt, pad_tgt]).reshape(NW * NCH, CH)
    tgt_c = jnp.concatenate(
        [tgt, jnp.zeros((pad,), jnp.int32)]).reshape(NW * NCH, CH)
    zeros = jnp.zeros((n_acc, D), jnp.float32)

    split = (112 * (NCH // 80), 48 * (NCH // 80)) if NCH % 80 == 0 else None
    parts = _agg_call(N, D, NCH, split)(x, src_p.reshape(-1, CH // 2),
                                        tgt_a.reshape(-1, CH // 2), zeros)
    h_pad = _dense(parts, x, W_neigh, W_self, b.reshape(1, D), n_acc)
    partial16 = _dot_call(N, D, NCH, n_acc)(h_pad, src_p, tgt_c)
    scores = _reduce16(partial16)
    return scores.reshape(-1)[:E]
